# Initial kernel scaffold; baseline (speedup 1.0000x reference)
import jax
import jax.numpy as jnp
from jax.experimental import pallas as pl


def kernel(boxes, scores):
    raise NotImplementedError("write your pallas kernel here")


# speedup vs baseline: 18.2223x; 18.2223x over previous
import jax
import jax.numpy as jnp
from jax.experimental import pallas as pl
from jax.experimental.pallas import tpu as pltpu

N = 20000
NPAD = 20480
ROWS = NPAD // 128
IMG = 1024.0
SCORE_THR = 0.05
IOU_THR = 0.5
MAX_DET = 1000
BIG = 2**30


def _nms_body(x1_ref, y1_ref, x2_ref, y2_ref, s_ref,
              d1_ref, d2_ref, d3_ref, d4_ref, d5_ref):
    x1 = x1_ref[...]
    y1 = y1_ref[...]
    x2 = x2_ref[...]
    y2 = y2_ref[...]
    raw = s_ref[...]

    areas = (x2 - x1 + 1.0) * (y2 - y1 + 1.0)
    lin = (jax.lax.broadcasted_iota(jnp.int32, (ROWS, 128), 0) * 128
           + jax.lax.broadcasted_iota(jnp.int32, (ROWS, 128), 1))
    scores0 = jnp.where(raw >= SCORE_THR, raw, -1.0)

    lin_out = (jax.lax.broadcasted_iota(jnp.int32, (8, 128), 0) * 128
               + jax.lax.broadcasted_iota(jnp.int32, (8, 128), 1))
    zeros8 = jnp.zeros((8, 128), jnp.float32)

    def step(i, carry):
        sw, d1, d2, d3, d4, d5 = carry
        m = jnp.max(sw)
        eligible = sw == m
        idx = jnp.min(jnp.where(eligible, lin, BIG))
        sel = lin == idx
        valid = m > 0.0

        bx1 = jnp.sum(jnp.where(sel, x1, 0.0))
        by1 = jnp.sum(jnp.where(sel, y1, 0.0))
        bx2 = jnp.sum(jnp.where(sel, x2, 0.0))
        by2 = jnp.sum(jnp.where(sel, y2, 0.0))
        barea = jnp.sum(jnp.where(sel, areas, 0.0))

        xmin = jnp.maximum(x1, bx1)
        ymin = jnp.maximum(y1, by1)
        xmax = jnp.minimum(x2, bx2)
        ymax = jnp.minimum(y2, by2)
        inter = (jnp.maximum(xmax - xmin, 0.0)
                 * jnp.maximum(ymax - ymin, 0.0))
        iou = inter / (barea + areas - inter)
        suppress = (iou > IOU_THR) & valid
        sw = jnp.where(suppress | sel, -1.0, sw)

        omask = (lin_out == i) & valid
        d1 = jnp.where(omask, bx1, d1)
        d2 = jnp.where(omask, by1, d2)
        d3 = jnp.where(omask, bx2, d3)
        d4 = jnp.where(omask, by2, d4)
        d5 = jnp.where(omask, m, d5)
        return sw, d1, d2, d3, d4, d5

    init = (scores0, zeros8, zeros8, zeros8, zeros8, zeros8)
    _, d1, d2, d3, d4, d5 = jax.lax.fori_loop(0, MAX_DET, step, init)
    d1_ref[...] = d1
    d2_ref[...] = d2
    d3_ref[...] = d3
    d4_ref[...] = d4
    d5_ref[...] = d5


def kernel(boxes, scores):
    pad = NPAD - N
    x1 = jnp.pad(boxes[:, 0], (0, pad)).reshape(ROWS, 128)
    y1 = jnp.pad(boxes[:, 1], (0, pad)).reshape(ROWS, 128)
    x2 = jnp.pad(boxes[:, 2], (0, pad)).reshape(ROWS, 128)
    y2 = jnp.pad(boxes[:, 3], (0, pad)).reshape(ROWS, 128)
    s = jnp.pad(scores, (0, pad), constant_values=-1.0).reshape(ROWS, 128)

    outs = pl.pallas_call(
        _nms_body,
        out_shape=[jax.ShapeDtypeStruct((8, 128), jnp.float32)] * 5,
        in_specs=[pl.BlockSpec(memory_space=pltpu.VMEM)] * 5,
        out_specs=[pl.BlockSpec(memory_space=pltpu.VMEM)] * 5,
    )(x1, y1, x2, y2, s)

    cols = [o.reshape(1024)[:MAX_DET] for o in outs]
    return jnp.stack(cols, axis=1)


# speedup vs baseline: 88.3159x; 4.8466x over previous
import jax
import jax.numpy as jnp
from jax.experimental import pallas as pl
from jax.experimental.pallas import tpu as pltpu

N = 20000
NPAD = 20480
ROWS = NPAD // 128
B = 128
NB = NPAD // B
KBUF = 1280
IOU_THR = 0.5
SCORE_THR = 0.05
MAX_DET = 1000

_DOT = dict(preferred_element_type=jnp.float32,
            precision=jax.lax.Precision.HIGHEST)


def _nms_body(rx1, ry1, rx2, ry2, rs, out_ref):
    out_ref[...] = jnp.zeros((KBUF, 8), jnp.float32)

    c_iota = jax.lax.broadcasted_iota(jnp.int32, (KBUF, 1), 0)
    jsub = jax.lax.broadcasted_iota(jnp.int32, (B, B), 0)
    ilane = jax.lax.broadcasted_iota(jnp.int32, (B, B), 1)
    tri = jsub < ilane
    u_incl = (jsub <= ilane).astype(jnp.float32)
    ident = (jsub == ilane).astype(jnp.float32)
    ones_row = jnp.ones((1, KBUF), jnp.float32)

    def tcol(v):
        return jax.lax.dot_general(ident, v, (((1,), (1,)), ((), ())), **_DOT)

    def cond(carry):
        b, kcnt, stop = carry
        return (b < NB) & (kcnt < MAX_DET) & jnp.logical_not(stop)

    def body(carry):
        b, kcnt, _ = carry
        x1r = rx1[pl.ds(b, 1), :]
        y1r = ry1[pl.ds(b, 1), :]
        x2r = rx2[pl.ds(b, 1), :]
        y2r = ry2[pl.ds(b, 1), :]
        sr = rs[pl.ds(b, 1), :]
        x1c = tcol(x1r)
        y1c = tcol(y1r)
        x2c = tcol(x2r)
        y2c = tcol(y2r)
        sc = tcol(sr)

        ar_r = (x2r - x1r + 1.0) * (y2r - y1r + 1.0)
        ar_c = (x2c - x1c + 1.0) * (y2c - y1c + 1.0)

        kx1 = out_ref[:, 0:1]
        ky1 = out_ref[:, 1:2]
        kx2 = out_ref[:, 2:3]
        ky2 = out_ref[:, 3:4]
        karea = (kx2 - kx1 + 1.0) * (ky2 - ky1 + 1.0)
        xmin = jnp.maximum(kx1, x1r)
        ymin = jnp.maximum(ky1, y1r)
        xmax = jnp.minimum(kx2, x2r)
        ymax = jnp.minimum(ky2, y2r)
        inter = (jnp.maximum(xmax - xmin, 0.0)
                 * jnp.maximum(ymax - ymin, 0.0))
        iou = inter / (karea + ar_r - inter)
        msup = (iou > IOU_THR).astype(jnp.float32)
        presup = jax.lax.dot_general(ones_row, msup,
                                     (((1,), (0,)), ((), ())), **_DOT)

        xmin2 = jnp.maximum(x1c, x1r)
        ymin2 = jnp.maximum(y1c, y1r)
        xmax2 = jnp.minimum(x2c, x2r)
        ymax2 = jnp.minimum(y2c, y2r)
        inter2 = (jnp.maximum(xmax2 - xmin2, 0.0)
                  * jnp.maximum(ymax2 - ymin2, 0.0))
        iou2 = inter2 / (ar_c + ar_r - inter2)
        smat = ((iou2 > IOU_THR) & tri).astype(jnp.float32)

        base = ((sr >= SCORE_THR) & (presup == 0.0)).astype(jnp.float32)

        def fcond(st):
            return st[1]

        def fbody(st):
            a, _ = st
            cnt = jax.lax.dot_general(a, smat,
                                      (((1,), (0,)), ((), ())), **_DOT)
            a_new = base * (cnt == 0.0).astype(jnp.float32)
            return a_new, jnp.any(a_new != a)

        alive, _ = jax.lax.while_loop(fcond, fbody, (base, True))

        incl = jax.lax.dot_general(alive, u_incl,
                                   (((1,), (0,)), ((), ())), **_DOT)
        pos = kcnt + incl.astype(jnp.int32) - alive.astype(jnp.int32)
        ponehot = ((c_iota == pos) & (alive > 0.0)).astype(jnp.float32)
        dblk = jnp.concatenate(
            [x1c, y1c, x2c, y2c, sc, jnp.zeros((B, 3), jnp.float32)],
            axis=1)
        out_ref[...] += jax.lax.dot_general(ponehot, dblk,
                                            (((1,), (0,)), ((), ())), **_DOT)

        kcnt = kcnt + jnp.sum(alive).astype(jnp.int32)
        stop = jnp.min(sr) < SCORE_THR
        return b + 1, kcnt, stop

    jax.lax.while_loop(cond, body, (jnp.int32(0), jnp.int32(0), False))


def kernel(boxes, scores):
    order = jnp.argsort(-scores, stable=True)
    sb = boxes[order]
    ss = scores[order]
    pad = NPAD - N

    def rows(v, cv=0.0):
        return jnp.pad(v, (0, pad), constant_values=cv).reshape(ROWS, 128)

    rx1 = rows(sb[:, 0])
    ry1 = rows(sb[:, 1])
    rx2 = rows(sb[:, 2])
    ry2 = rows(sb[:, 3])
    rs = rows(ss, cv=-1.0)

    out = pl.pallas_call(
        _nms_body,
        out_shape=jax.ShapeDtypeStruct((KBUF, 8), jnp.float32),
        in_specs=[pl.BlockSpec(memory_space=pltpu.VMEM)] * 5,
        out_specs=pl.BlockSpec(memory_space=pltpu.VMEM),
    )(rx1, ry1, rx2, ry2, rs)

    return out[:MAX_DET, :5]
